# baseline (speedup 1.0000x reference)
import jax
import jax.numpy as jnp
from jax.experimental import pallas as pl


def _copy_body(src_ref, dst_ref):
    dst_ref[...] = src_ref[...]


def _bank_snapshot(bank):
    dim, size = bank.shape
    blk = 8192
    grid = size // blk
    return pl.pallas_call(
        _copy_body,
        grid=(grid,),
        in_specs=[pl.BlockSpec((dim, blk), lambda i: (0, i))],
        out_specs=pl.BlockSpec((dim, blk), lambda i: (0, i)),
        out_shape=jax.ShapeDtypeStruct(bank.shape, bank.dtype),
    )(bank)


def kernel(output, bank):
    return (output, _bank_snapshot(bank))
